# baseline (speedup 1.0000x reference)
import functools

import jax
import jax.numpy as jnp
from jax import lax
from jax.experimental import pallas as pl
from jax.experimental.pallas import tpu as pltpu
from jax.experimental.pallas import tpu_sc as plsc

EMBED = 16
BATCH = 16384
ALPHA = 0.5

NC = 2
NS = 16
NW = NC * NS
BPW = BATCH // NW


TAIL = 64
SPW = BPW - TAIL
B_TC = NW * TAIL
RING = 4
NBANK = 2
NBATCH = SPW // RING
NPAIR = NBATCH // NBANK
LANE = 128


def _sc_gather_body(uidx_hbm, iidx_hbm, gu_t, gi_t, mu_t, mi_t,
                    gu_o, gi_o, mu_o, mi_o,
                    uidx_s, iidx_s,
                    gu_r, gi_r, mu_r, mi_r,
                    gu_v, gi_v, mu_v, mi_v,
                    s00, s01, s10, s11, s20, s21, s30, s31):
    wid = lax.axis_index("s") * NC + lax.axis_index("c")
    base = wid * BPW
    pltpu.sync_copy(uidx_hbm.at[wid], uidx_s.at[pl.ds(0, BPW)])
    pltpu.sync_copy(iidx_hbm.at[wid], iidx_s.at[pl.ds(0, BPW)])

    sems = ((s00, s01), (s10, s11), (s20, s21), (s30, s31))
    rings = (gu_r, gi_r, mu_r, mi_r)
    tabs = (gu_t, gi_t, mu_t, mi_t)
    outsv = (gu_v, gi_v, mu_v, mi_v)
    iota16 = lax.iota(jnp.int32, 16)

    def load_idx(gg):
        uvec = uidx_s[pl.ds(gg * NBANK * RING, 16)]
        ivec = iidx_s[pl.ds(gg * NBANK * RING, 16)]
        return uvec, ivec

    def enqueue(bank, uvec, ivec, lane_off):
        for r in range(RING):
            uu = uvec[lane_off + r]
            ii = ivec[lane_off + r]
            for t in range(4):
                idx = uu if t % 2 == 0 else ii
                col = idx & jnp.int32(-LANE)
                src = tabs[t].at[:, pl.ds(pl.multiple_of(col, LANE), LANE)]
                pltpu.async_copy(src, rings[t].at[bank * RING + r],
                                 sems[t][bank])

    def drain(bank):
        for r in range(RING):
            for t in range(4):
                pltpu.make_async_copy(tabs[t].at[:, pl.ds(0, LANE)],
                                      rings[t].at[bank * RING + r],
                                      sems[t][bank]).wait()

    def extract(bank, g, uvec, ivec, lane_off):
        for r in range(RING):
            b = g * RING + r
            for t in range(4):
                u = uvec[lane_off + r] if t % 2 == 0 else ivec[lane_off + r]
                lane = u & jnp.int32(LANE - 1)
                vec = plsc.load_gather(
                    rings[t],
                    [jnp.full((16,), bank * RING + r, jnp.int32), iota16,
                     jnp.full((16,), lane, jnp.int32)])
                plsc.store_scatter(outsv[t],
                                   [iota16, jnp.full((16,), b, jnp.int32)],
                                   vec)

    uvec0, ivec0 = load_idx(0)
    enqueue(0, uvec0, ivec0, 0)

    def outer(gg, _):
        g0 = gg * NBANK
        uvec, ivec = load_idx(gg)
        nvec, jvec = load_idx(gg + 1)
        enqueue(1, uvec, ivec, RING)
        drain(0)
        extract(0, g0, uvec, ivec, 0)

        @pl.when(gg + 1 < NPAIR)
        def _():
            enqueue(0, nvec, jvec, 0)

        drain(1)
        extract(1, g0 + 1, uvec, ivec, RING)
        return ()

    lax.fori_loop(0, NPAIR, outer, ())

    out_sl = pl.ds(base, BPW)
    pltpu.sync_copy(gu_v, gu_o.at[:, out_sl])
    pltpu.sync_copy(gi_v, gi_o.at[:, out_sl])
    pltpu.sync_copy(mu_v, mu_o.at[:, out_sl])
    pltpu.sync_copy(mi_v, mi_o.at[:, out_sl])


_sc_gather = functools.partial(
    pl.kernel,
    out_type=[jax.ShapeDtypeStruct((EMBED, BATCH), jnp.float32)] * 4,
    mesh=plsc.VectorSubcoreMesh(core_axis_name="c", subcore_axis_name="s"),
    scratch_types=[
        pltpu.VMEM((BPW,), jnp.int32),
        pltpu.VMEM((BPW,), jnp.int32),
        pltpu.VMEM((NBANK * RING, EMBED, LANE), jnp.float32),
        pltpu.VMEM((NBANK * RING, EMBED, LANE), jnp.float32),
        pltpu.VMEM((NBANK * RING, EMBED, LANE), jnp.float32),
        pltpu.VMEM((NBANK * RING, EMBED, LANE), jnp.float32),
        pltpu.VMEM((EMBED, BPW), jnp.float32),
        pltpu.VMEM((EMBED, BPW), jnp.float32),
        pltpu.VMEM((EMBED, BPW), jnp.float32),
        pltpu.VMEM((EMBED, BPW), jnp.float32),
        pltpu.SemaphoreType.DMA,
        pltpu.SemaphoreType.DMA,
        pltpu.SemaphoreType.DMA,
        pltpu.SemaphoreType.DMA,
        pltpu.SemaphoreType.DMA,
        pltpu.SemaphoreType.DMA,
        pltpu.SemaphoreType.DMA,
        pltpu.SemaphoreType.DMA,
    ],
    compiler_params=pltpu.CompilerParams(use_tc_tiling_on_sc=True,
                                         needs_layout_passes=False),
)(_sc_gather_body)


G = 8


def _tc_gather_body(uref, iref, *args):
    blocks = args[:4 * G]
    outs = args[4 * G:]
    i = pl.program_id(0)
    lane_iota = lax.broadcasted_iota(jnp.int32, (EMBED, LANE), 1)
    for t in range(4):
        idx_ref = uref if t % 2 == 0 else iref
        cols = []
        for k in range(G):
            lane = idx_ref[i * G + k] % LANE
            blk = blocks[t * G + k][...]
            sel = jnp.where(lane_iota == lane, blk, 0.0)
            cols.append(jnp.sum(sel, axis=1, keepdims=True))
        outs[t][...] = jnp.concatenate(cols, axis=1)[None]


def _tc_gather(gu_t, gi_t, mu_t, mi_t, uidx, iidx):
    def make_spec(t, k):
        def imap(i, uref, iref):
            r = uref if t % 2 == 0 else iref
            return (0, r[i * G + k] // LANE)
        return pl.BlockSpec((EMBED, LANE), imap)

    in_specs = [make_spec(t, k) for t in range(4) for k in range(G)]
    out_specs = [pl.BlockSpec((1, EMBED, G),
                              lambda i, uref, iref: (i, 0, 0))] * 4
    outs = pl.pallas_call(
        _tc_gather_body,
        grid_spec=pltpu.PrefetchScalarGridSpec(
            num_scalar_prefetch=2,
            grid=(B_TC // G,),
            in_specs=in_specs,
            out_specs=out_specs,
        ),
        out_shape=[jax.ShapeDtypeStruct((B_TC // G, EMBED, G),
                                        jnp.float32)] * 4,
    )(uidx, iidx,
      *([gu_t] * G + [gi_t] * G + [mu_t] * G + [mi_t] * G))
    return [o.transpose(1, 0, 2).reshape(EMBED, B_TC) for o in outs]


BLK = 2048


def _tc_mlp_body(gu, gi, mu, mi, w0, b0, w1, b1, w2, b2, w3, b3, wp, bp, out):
    f32 = jnp.float32
    dims = (((0,), (0,)), ((), ()))
    h = jnp.maximum(
        lax.dot_general(w0[0:EMBED, :], mu[...], dims, preferred_element_type=f32)
        + lax.dot_general(w0[EMBED:2 * EMBED, :], mi[...], dims,
                          preferred_element_type=f32)
        + b0[...], 0.0)
    h = jnp.maximum(
        lax.dot_general(w1[...], h, dims, preferred_element_type=f32) + b1[...], 0.0)
    h = jnp.maximum(
        lax.dot_general(w2[...], h, dims, preferred_element_type=f32) + b2[...], 0.0)
    h = jnp.maximum(
        lax.dot_general(w3[...], h, dims, preferred_element_type=f32) + b3[...], 0.0)
    gmf = gu[...] * gi[...]
    pred = (ALPHA * lax.dot_general(wp[0:EMBED, :], gmf, dims,
                                    preferred_element_type=f32)
            + (1.0 - ALPHA) * lax.dot_general(wp[EMBED:, :], h, dims,
                                              preferred_element_type=f32)
            + bp[...])
    out[...] = pred


def _tc_mlp(gu, gi, mu, mi, w0, b0, w1, b1, w2, b2, w3, b3, wp, bp):
    nb = BATCH // BLK
    col_spec = pl.BlockSpec((EMBED, BLK), lambda i: (0, i))

    def full(a):
        return pl.BlockSpec(a.shape, lambda i: tuple(0 for _ in a.shape))

    return pl.pallas_call(
        _tc_mlp_body,
        grid=(nb,),
        in_specs=[col_spec, col_spec, col_spec, col_spec,
                  full(w0), full(b0), full(w1), full(b1),
                  full(w2), full(b2), full(w3), full(b3),
                  full(wp), full(bp)],
        out_specs=pl.BlockSpec((1, BLK), lambda i: (0, i)),
        out_shape=jax.ShapeDtypeStruct((1, BATCH), jnp.float32),
    )(gu, gi, mu, mi, w0, b0, w1, b1, w2, b2, w3, b3, wp, bp)


def kernel(user_input, item_input, gmf_user_table, gmf_item_table,
           mlp_user_table, mlp_item_table,
           W0, b0, W1, b1, W2, b2, W3, b3, Wp, bp):
    uidx = user_input.astype(jnp.int32).reshape(NW, BPW)
    iidx = item_input.astype(jnp.int32).reshape(NW, BPW)
    tabs = (gmf_user_table.T, gmf_item_table.T,
            mlp_user_table.T, mlp_item_table.T)
    sc_out = _sc_gather(uidx, iidx, *tabs)
    tc_out = _tc_gather(*tabs, uidx[:, SPW:].reshape(-1),
                        iidx[:, SPW:].reshape(-1))
    def stitch(sc, tc):
        sc3 = sc.reshape(EMBED, NW, BPW)[:, :, :SPW]
        tc3 = tc.reshape(EMBED, NW, TAIL)
        return jnp.concatenate([sc3, tc3], axis=2).reshape(EMBED, BATCH)

    gu, gi, mu, mi = (stitch(s, t) for s, t in zip(sc_out, tc_out))
    pred_t = _tc_mlp(gu, gi, mu, mi,
                     W0, b0.reshape(-1, 1), W1, b1.reshape(-1, 1),
                     W2, b2.reshape(-1, 1), W3, b3.reshape(-1, 1),
                     Wp, bp.reshape(1, 1))
    return pred_t.reshape(BATCH, 1)


# speedup vs baseline: 1.5391x; 1.5391x over previous
import functools

import jax
import jax.numpy as jnp
from jax import lax
from jax.experimental import pallas as pl
from jax.experimental.pallas import tpu as pltpu
from jax.experimental.pallas import tpu_sc as plsc

EMBED = 16
BATCH = 16384
ALPHA = 0.5

NC = 2
NS = 16
NW = NC * NS
BPW = BATCH // NW


RING = 4
NBANK = 2
NBATCH = BPW // RING
NPAIR = NBATCH // NBANK
LANE = 128


def _sc_gather_body(uidx_hbm, iidx_hbm, gu_t, gi_t, mu_t, mi_t,
                    gu_o, gi_o, mu_o, mi_o,
                    uidx_s, iidx_s,
                    gu_r, gi_r, mu_r, mi_r,
                    gu_v, gi_v, mu_v, mi_v,
                    s00, s01, s10, s11, s20, s21, s30, s31):
    wid = lax.axis_index("s") * NC + lax.axis_index("c")
    base = wid * BPW
    pltpu.sync_copy(uidx_hbm.at[wid], uidx_s.at[pl.ds(0, BPW)])
    pltpu.sync_copy(iidx_hbm.at[wid], iidx_s.at[pl.ds(0, BPW)])

    sems = ((s00, s01), (s10, s11), (s20, s21), (s30, s31))
    rings = (gu_r, gi_r, mu_r, mi_r)
    tabs = (gu_t, gi_t, mu_t, mi_t)
    outsv = (gu_v, gi_v, mu_v, mi_v)
    iota16 = lax.iota(jnp.int32, 16)

    def load_idx(gg):
        uvec = uidx_s[pl.ds(gg * NBANK * RING, 16)]
        ivec = iidx_s[pl.ds(gg * NBANK * RING, 16)]
        return uvec, ivec

    def enqueue(bank, uvec, ivec, lane_off):
        for r in range(RING):
            uu = uvec[lane_off + r]
            ii = ivec[lane_off + r]
            for t in range(4):
                idx = uu if t % 2 == 0 else ii
                col = idx & jnp.int32(-LANE)
                src = tabs[t].at[:, pl.ds(pl.multiple_of(col, LANE), LANE)]
                pltpu.async_copy(src, rings[t].at[bank * RING + r],
                                 sems[t][bank])

    def drain(bank):
        for r in range(RING):
            for t in range(4):
                pltpu.make_async_copy(tabs[t].at[:, pl.ds(0, LANE)],
                                      rings[t].at[bank * RING + r],
                                      sems[t][bank]).wait()

    def extract(bank, g, uvec, ivec, lane_off):
        for r in range(RING):
            b = g * RING + r
            for t in range(4):
                u = uvec[lane_off + r] if t % 2 == 0 else ivec[lane_off + r]
                lane = u & jnp.int32(LANE - 1)
                vec = plsc.load_gather(
                    rings[t],
                    [jnp.full((16,), bank * RING + r, jnp.int32), iota16,
                     jnp.full((16,), lane, jnp.int32)])
                plsc.store_scatter(outsv[t],
                                   [iota16, jnp.full((16,), b, jnp.int32)],
                                   vec)

    uvec0, ivec0 = load_idx(0)
    enqueue(0, uvec0, ivec0, 0)

    def outer(gg, _):
        g0 = gg * NBANK
        uvec, ivec = load_idx(gg)
        nvec, jvec = load_idx(gg + 1)
        enqueue(1, uvec, ivec, RING)
        drain(0)
        extract(0, g0, uvec, ivec, 0)

        @pl.when(gg + 1 < NPAIR)
        def _():
            enqueue(0, nvec, jvec, 0)

        drain(1)
        extract(1, g0 + 1, uvec, ivec, RING)
        return ()

    lax.fori_loop(0, NPAIR, outer, ())

    out_sl = pl.ds(base, BPW)
    pltpu.sync_copy(gu_v, gu_o.at[:, out_sl])
    pltpu.sync_copy(gi_v, gi_o.at[:, out_sl])
    pltpu.sync_copy(mu_v, mu_o.at[:, out_sl])
    pltpu.sync_copy(mi_v, mi_o.at[:, out_sl])


_sc_gather = functools.partial(
    pl.kernel,
    out_type=[jax.ShapeDtypeStruct((EMBED, BATCH), jnp.float32)] * 4,
    mesh=plsc.VectorSubcoreMesh(core_axis_name="c", subcore_axis_name="s"),
    scratch_types=[
        pltpu.VMEM((BPW + 16,), jnp.int32),
        pltpu.VMEM((BPW + 16,), jnp.int32),
        pltpu.VMEM((NBANK * RING, EMBED, LANE), jnp.float32),
        pltpu.VMEM((NBANK * RING, EMBED, LANE), jnp.float32),
        pltpu.VMEM((NBANK * RING, EMBED, LANE), jnp.float32),
        pltpu.VMEM((NBANK * RING, EMBED, LANE), jnp.float32),
        pltpu.VMEM((EMBED, BPW), jnp.float32),
        pltpu.VMEM((EMBED, BPW), jnp.float32),
        pltpu.VMEM((EMBED, BPW), jnp.float32),
        pltpu.VMEM((EMBED, BPW), jnp.float32),
        pltpu.SemaphoreType.DMA,
        pltpu.SemaphoreType.DMA,
        pltpu.SemaphoreType.DMA,
        pltpu.SemaphoreType.DMA,
        pltpu.SemaphoreType.DMA,
        pltpu.SemaphoreType.DMA,
        pltpu.SemaphoreType.DMA,
        pltpu.SemaphoreType.DMA,
    ],
    compiler_params=pltpu.CompilerParams(use_tc_tiling_on_sc=True,
                                         needs_layout_passes=False),
)(_sc_gather_body)


BLK = 2048


def _tc_mlp_body(gu, gi, mu, mi, w0, b0, w1, b1, w2, b2, w3, b3, wp, bp, out):
    f32 = jnp.float32
    dims = (((0,), (0,)), ((), ()))
    h = jnp.maximum(
        lax.dot_general(w0[0:EMBED, :], mu[...], dims, preferred_element_type=f32)
        + lax.dot_general(w0[EMBED:2 * EMBED, :], mi[...], dims,
                          preferred_element_type=f32)
        + b0[...], 0.0)
    h = jnp.maximum(
        lax.dot_general(w1[...], h, dims, preferred_element_type=f32) + b1[...], 0.0)
    h = jnp.maximum(
        lax.dot_general(w2[...], h, dims, preferred_element_type=f32) + b2[...], 0.0)
    h = jnp.maximum(
        lax.dot_general(w3[...], h, dims, preferred_element_type=f32) + b3[...], 0.0)
    gmf = gu[...] * gi[...]
    pred = (ALPHA * lax.dot_general(wp[0:EMBED, :], gmf, dims,
                                    preferred_element_type=f32)
            + (1.0 - ALPHA) * lax.dot_general(wp[EMBED:, :], h, dims,
                                              preferred_element_type=f32)
            + bp[...])
    out[...] = pred


def _tc_mlp(gu, gi, mu, mi, w0, b0, w1, b1, w2, b2, w3, b3, wp, bp):
    nb = BATCH // BLK
    col_spec = pl.BlockSpec((EMBED, BLK), lambda i: (0, i))

    def full(a):
        return pl.BlockSpec(a.shape, lambda i: tuple(0 for _ in a.shape))

    return pl.pallas_call(
        _tc_mlp_body,
        grid=(nb,),
        in_specs=[col_spec, col_spec, col_spec, col_spec,
                  full(w0), full(b0), full(w1), full(b1),
                  full(w2), full(b2), full(w3), full(b3),
                  full(wp), full(bp)],
        out_specs=pl.BlockSpec((1, BLK), lambda i: (0, i)),
        out_shape=jax.ShapeDtypeStruct((1, BATCH), jnp.float32),
    )(gu, gi, mu, mi, w0, b0, w1, b1, w2, b2, w3, b3, wp, bp)


def kernel(user_input, item_input, gmf_user_table, gmf_item_table,
           mlp_user_table, mlp_item_table,
           W0, b0, W1, b1, W2, b2, W3, b3, Wp, bp):
    uidx = user_input.astype(jnp.int32).reshape(NW, BPW)
    iidx = item_input.astype(jnp.int32).reshape(NW, BPW)
    gu, gi, mu, mi = _sc_gather(uidx, iidx,
                                gmf_user_table.T, gmf_item_table.T,
                                mlp_user_table.T, mlp_item_table.T)
    pred_t = _tc_mlp(gu, gi, mu, mi,
                     W0, b0.reshape(-1, 1), W1, b1.reshape(-1, 1),
                     W2, b2.reshape(-1, 1), W3, b3.reshape(-1, 1),
                     Wp, bp.reshape(1, 1))
    return pred_t.reshape(BATCH, 1)
